# baseline (speedup 1.0000x reference)
import functools

import jax
import jax.numpy as jnp
from jax import lax
from jax.experimental import pallas as pl
from jax.experimental.pallas import tpu as pltpu
from jax.experimental.pallas import tpu_sc as plsc

N = 10000
E = 320000
H = 32
EMB = 32
NC = 2
NS = 16
NTILES = NC * NS
FPT = H // NS
EPT = E // NTILES
HALF = E // NC
CH = 8000
NCHUNK = HALF // CH
CHD = 2000
BN_SCALE = 1.0 / (1.0 + 1e-5) ** 0.5


def _pack_g(g32):
    lo = lax.bitcast_convert_type(g32[:16].astype(jnp.bfloat16),
                                  jnp.uint16).astype(jnp.uint32)
    hi = lax.bitcast_convert_type(g32[16:].astype(jnp.bfloat16),
                                  jnp.uint16).astype(jnp.uint32)
    return lax.bitcast_convert_type(lo | (hi << 16), jnp.int32)


def _prologue_body(ea_ref, w_ref, r_ref, c_ref, x_ref, emb_ref,
                   wf_ref, bf_ref, wc_ref, bc_ref, ew_ref, rc_ref, h0t_ref):
    jr = lax.broadcasted_iota(jnp.int32, (256, 128), 0)
    ji = lax.broadcasted_iota(jnp.int32, (256, 128), 1)
    m = (jnp.where(jr == 2 * ji, w_ref[0, 0], 0.0)
         + jnp.where(jr == 2 * ji + 1, w_ref[1, 0], 0.0))
    ew = ea_ref[...] @ m + w_ref[2, 0]
    mx = jnp.max(ew)
    ew_ref[...] = jnp.where(mx > 1000.0, ew / mx, ew)
    rc_ref[...] = r_ref[...] | (c_ref[...] << 16)

    xc = jnp.nan_to_num(x_ref[...], nan=0.0)
    feat = xc @ wf_ref[...] + bf_ref[...]
    wtop = wc_ref[...][:EMB]
    wbot = wc_ref[...][EMB:]
    t1 = lax.dot_general(wtop, emb_ref[...], (((0,), (1,)), ((), ())))
    t2 = lax.dot_general(wbot, feat, (((0,), (1,)), ((), ())))
    h0t_ref[...] = jax.nn.relu(t1 + t2 + bc_ref[...])


def _norm_body(part_ref, h0t_ref, dis_ref, sn_ref, g0p_ref):
    deg = 1.0 + jnp.sum(part_ref[...], axis=0, keepdims=True)
    safe = jnp.where(deg > 0, deg, 1.0)
    dis = jnp.where(deg > 0, 1.0 / jnp.sqrt(safe), 0.0)
    dis_ref[...] = dis
    sn_ref[...] = dis * dis
    g0p_ref[...] = _pack_g(h0t_ref[...] * dis)


def _unpack_p(p_ref):
    psum = p_ref[0] + p_ref[1]
    return jnp.concatenate([psum[:, 0], psum[:, 1]], axis=0)


def _layer_body(p_ref, ht_ref, dis_ref, sn_ref, w_ref, b_ref, gam_ref,
                bet_ref, hout_ref, gout_ref):
    p = _unpack_p(p_ref)
    ht = ht_ref[...]
    dis = dis_ref[...]
    agg = dis * p + sn_ref[...] * ht
    outt = lax.dot_general(w_ref[...], agg, (((0,), (0,)), ((), ())))
    outt = outt + b_ref[...]
    hn = jax.nn.relu(outt) * (BN_SCALE * gam_ref[...]) + bet_ref[...] + ht
    hout_ref[...] = hn
    gout_ref[...] = _pack_g(hn * dis)


def _layer_final_body(p_ref, ht_ref, dis_ref, sn_ref, w_ref, b_ref, gam_ref,
                      bet_ref, wl_ref, bl_ref, out_ref):
    p = _unpack_p(p_ref)
    ht = ht_ref[...]
    dis = dis_ref[...]
    agg = dis * p + sn_ref[...] * ht
    outt = lax.dot_general(w_ref[...], agg, (((0,), (0,)), ((), ())))
    outt = outt + b_ref[...]
    hn = jax.nn.relu(outt) * (BN_SCALE * gam_ref[...]) + bet_ref[...] + ht
    o = lax.dot_general(hn, wl_ref[...], (((0,), (0,)), ((), ())))
    out_ref[...] = jnp.clip(o + bl_ref[...], -10.0, 10.0)


def _tc(body, out_shape, *args):
    return pl.pallas_call(body, out_shape=out_shape)(*args)


_MESH = plsc.VectorSubcoreMesh(core_axis_name="c", subcore_axis_name="s")
_SC_PARAMS = pltpu.CompilerParams(needs_layout_passes=False)


@functools.partial(
    pl.kernel,
    out_type=jax.ShapeDtypeStruct((NTILES * N,), jnp.float32),
    mesh=_MESH,
    compiler_params=_SC_PARAMS,
    scratch_types=[
        pltpu.VMEM((CHD,), jnp.float32),
        pltpu.VMEM((CHD,), jnp.int32),
        pltpu.VMEM((N,), jnp.float32),
    ],
)
def _sc_deg(ew_hbm, col_hbm, part_hbm, ew_v, col_v, acc_v):
    cid = lax.axis_index("c")
    sid = lax.axis_index("s")
    wid = cid * NS + sid
    base = wid * EPT

    @plsc.parallel_loop(0, N // 16, unroll=4)
    def zero_body(i):
        acc_v[pl.ds(i * 16, 16)] = jnp.zeros((16,), jnp.float32)

    def chunk_body(k, _):
        off = base + k * CHD
        pltpu.sync_copy(ew_hbm.at[pl.ds(off, CHD)], ew_v)
        pltpu.sync_copy(col_hbm.at[pl.ds(off, CHD)], col_v)

        @plsc.parallel_loop(0, CHD // 16, unroll=4)
        def grp(j):
            c16 = col_v[pl.ds(j * 16, 16)]
            w16 = ew_v[pl.ds(j * 16, 16)]
            plsc.addupdate_scatter(acc_v, [c16], w16)

        return _

    lax.fori_loop(0, EPT // CHD, chunk_body, None)
    pltpu.sync_copy(acc_v, part_hbm.at[pl.ds(wid * N, N)])


@functools.partial(
    pl.kernel,
    out_type=jax.ShapeDtypeStruct((NC * H * N,), jnp.float32),
    mesh=_MESH,
    compiler_params=_SC_PARAMS,
    scratch_types=[
        pltpu.VMEM((N,), jnp.int32),
        pltpu.VMEM((FPT * N,), jnp.float32),
        pltpu.VMEM((CH,), jnp.int32),
        pltpu.VMEM((CH,), jnp.int32),
        pltpu.VMEM((CH,), jnp.float32),
        pltpu.VMEM((CH,), jnp.float32),
        pltpu.SemaphoreType.DMA,
        pltpu.SemaphoreType.DMA,
        pltpu.SemaphoreType.DMA,
        pltpu.SemaphoreType.DMA,
        pltpu.SemaphoreType.DMA,
    ],
)
def _sc_scatter(gp_hbm, rc_hbm, w_hbm, p_hbm, g_v, acc_v, rc_v0, rc_v1,
                w_v0, w_v1, rsem0, rsem1, wsem0, wsem1, gsem):
    cid = lax.axis_index("c")
    sid = lax.axis_index("s")
    rsems = (rsem0, rsem1)
    wsems = (wsem0, wsem1)
    rcbufs = (rc_v0, rc_v1)
    wbufs = (w_v0, w_v1)
    gcp = pltpu.async_copy(gp_hbm.at[pl.ds(sid * N, N)], g_v, gsem)

    @plsc.parallel_loop(0, (FPT * N) // 16, unroll=4)
    def zero_body(i):
        acc_v[pl.ds(i * 16, 16)] = jnp.zeros((16,), jnp.float32)

    gcp.wait()
    ebase = cid * HALF

    def issue(k, b):
        pltpu.async_copy(rc_hbm.at[pl.ds(ebase + k * CH, CH)],
                         rcbufs[b], rsems[b])
        pltpu.async_copy(w_hbm.at[pl.ds(ebase + k * CH, CH)],
                         wbufs[b], wsems[b])

    issue(0, 0)
    issue(1, 1)

    def outer(gi, _):
        for b in range(2):
            rcbuf = rcbufs[b]
            wbuf = wbufs[b]
            k = gi * 2 + b
            pltpu.make_async_copy(rc_hbm.at[pl.ds(ebase, CH)],
                                  rcbuf, rsems[b]).wait()
            pltpu.make_async_copy(w_hbm.at[pl.ds(ebase, CH)],
                                  wbuf, wsems[b]).wait()

            @plsc.parallel_loop(0, CH // 16, unroll=8)
            def grp(j):
                rc16 = rcbuf[pl.ds(j * 16, 16)]
                w16 = wbuf[pl.ds(j * 16, 16)]
                r16 = rc16 & 0xFFFF
                c16 = lax.shift_right_logical(rc16, 16)
                vals = plsc.load_gather(g_v, [r16])
                v0 = plsc.bitcast(lax.shift_left(vals, 16), jnp.float32)
                v1 = plsc.bitcast(vals & jnp.int32(-65536), jnp.float32)
                plsc.addupdate_scatter(acc_v, [c16], v0 * w16)
                plsc.addupdate_scatter(acc_v, [c16 + N], v1 * w16)

            @pl.when(k + 2 < NCHUNK)
            def _reissue():
                issue(k + 2, b)

        return _

    lax.fori_loop(0, NCHUNK // 2, outer, None)
    pltpu.sync_copy(acc_v, p_hbm.at[pl.ds(cid * (H * N) + sid * (FPT * N),
                                          FPT * N)])


def kernel(x, edge_index, edge_attr, emb_w, W_feat, b_feat, W_edge, b_edge,
           W_comb, b_comb, W_c0, b_c0, W_c1, b_c1, W_c2, b_c2, gamma, beta,
           W_lin, b_lin):
    f32 = jnp.float32
    row = edge_index[0]
    col = edge_index[1]
    ea = edge_attr.reshape(E // 128, 256)
    wpack = jnp.concatenate([W_edge, b_edge[:, None]], axis=0)

    r2 = row.reshape(E // 128, 128)
    c2 = col.reshape(E // 128, 128)
    ew2, rc2, h0t = _tc(
        _prologue_body,
        (jax.ShapeDtypeStruct((E // 128, 128), f32),
         jax.ShapeDtypeStruct((E // 128, 128), jnp.int32),
         jax.ShapeDtypeStruct((H, N), f32)),
        ea, wpack, r2, c2, x, emb_w, W_feat, b_feat.reshape(1, EMB),
        W_comb, b_comb.reshape(H, 1))
    ew = ew2.reshape(E)
    rc = rc2.reshape(E)

    part = _sc_deg(ew, col).reshape(NTILES, N)

    dis, sn, gp = _tc(
        _norm_body,
        (jax.ShapeDtypeStruct((1, N), f32),
         jax.ShapeDtypeStruct((1, N), f32),
         jax.ShapeDtypeStruct((NS, N), jnp.int32)),
        part, h0t)

    ht = h0t
    for (W, b) in ((W_c0, b_c0), (W_c1, b_c1)):
        pflat = _sc_scatter(gp.reshape(NS * N), rc, ew)
        p2 = pflat.reshape(NC, NS, FPT, N)
        ht, gp = _tc(
            _layer_body,
            (jax.ShapeDtypeStruct((H, N), f32),
             jax.ShapeDtypeStruct((NS, N), jnp.int32)),
            p2, ht, dis, sn, W, b.reshape(H, 1), gamma.reshape(H, 1),
            beta.reshape(H, 1))

    pflat = _sc_scatter(gp.reshape(NS * N), rc, ew)
    p2 = pflat.reshape(NC, NS, FPT, N)
    out = _tc(
        _layer_final_body, jax.ShapeDtypeStruct((N, 1), f32),
        p2, ht, dis, sn, W_c2, b_c2.reshape(H, 1), gamma.reshape(H, 1),
        beta.reshape(H, 1), W_lin, b_lin.reshape(1, 1))
    return out


# speedup vs baseline: 1.5830x; 1.5830x over previous
import functools

import jax
import jax.numpy as jnp
from jax import lax
from jax.experimental import pallas as pl
from jax.experimental.pallas import tpu as pltpu
from jax.experimental.pallas import tpu_sc as plsc

N = 10000
E = 320000
H = 32
EMB = 32
NC = 2
NS = 16
NTILES = NC * NS
FPT = H // NS
EPT = E // NTILES
HALF = E // NC
CH = 8000
NCHUNK = HALF // CH
CHD = 2000
BN_SCALE = 1.0 / (1.0 + 1e-5) ** 0.5


def _pack_g(g32):
    lo = lax.bitcast_convert_type(g32[:16].astype(jnp.bfloat16),
                                  jnp.uint16).astype(jnp.uint32)
    hi = lax.bitcast_convert_type(g32[16:].astype(jnp.bfloat16),
                                  jnp.uint16).astype(jnp.uint32)
    return lax.bitcast_convert_type(lo | (hi << 16), jnp.int32)


def _prologue_body(a0_ref, a1_ref, w_ref, r_ref, c_ref, x_ref, emb_ref,
                   wf_ref, bf_ref, wc_ref, bc_ref, ew_ref, rc_ref, h0t_ref):
    a0 = a0_ref[...]
    a1 = a1_ref[...]
    ew = a0 * w_ref[0, 0] + a1 * w_ref[1, 0] + w_ref[2, 0]
    mx = jnp.max(ew)
    ew_ref[...] = jnp.where(mx > 1000.0, ew / mx, ew)
    rc_ref[...] = r_ref[...] | (c_ref[...] << 16)

    xc = jnp.nan_to_num(x_ref[...], nan=0.0)
    feat = xc @ wf_ref[...] + bf_ref[...]
    wtop = wc_ref[...][:EMB]
    wbot = wc_ref[...][EMB:]
    t1 = lax.dot_general(wtop, emb_ref[...], (((0,), (1,)), ((), ())))
    t2 = lax.dot_general(wbot, feat, (((0,), (1,)), ((), ())))
    h0t_ref[...] = jax.nn.relu(t1 + t2 + bc_ref[...])


def _norm_body(part_ref, h0t_ref, dis_ref, sn_ref, g0p_ref):
    deg = 1.0 + jnp.sum(part_ref[...], axis=0, keepdims=True)
    safe = jnp.where(deg > 0, deg, 1.0)
    dis = jnp.where(deg > 0, 1.0 / jnp.sqrt(safe), 0.0)
    dis_ref[...] = dis
    sn_ref[...] = dis * dis
    g0p_ref[...] = _pack_g(h0t_ref[...] * dis)


def _unpack_p(p_ref):
    psum = p_ref[0] + p_ref[1]
    return jnp.concatenate([psum[:, 0], psum[:, 1]], axis=0)


def _layer_body(p_ref, ht_ref, dis_ref, sn_ref, w_ref, b_ref, gam_ref,
                bet_ref, hout_ref, gout_ref):
    p = _unpack_p(p_ref)
    ht = ht_ref[...]
    dis = dis_ref[...]
    agg = dis * p + sn_ref[...] * ht
    outt = lax.dot_general(w_ref[...], agg, (((0,), (0,)), ((), ())))
    outt = outt + b_ref[...]
    hn = jax.nn.relu(outt) * (BN_SCALE * gam_ref[...]) + bet_ref[...] + ht
    hout_ref[...] = hn
    gout_ref[...] = _pack_g(hn * dis)


def _layer_final_body(p_ref, ht_ref, dis_ref, sn_ref, w_ref, b_ref, gam_ref,
                      bet_ref, wl_ref, bl_ref, out_ref):
    p = _unpack_p(p_ref)
    ht = ht_ref[...]
    dis = dis_ref[...]
    agg = dis * p + sn_ref[...] * ht
    outt = lax.dot_general(w_ref[...], agg, (((0,), (0,)), ((), ())))
    outt = outt + b_ref[...]
    hn = jax.nn.relu(outt) * (BN_SCALE * gam_ref[...]) + bet_ref[...] + ht
    o = lax.dot_general(hn, wl_ref[...], (((0,), (0,)), ((), ())))
    out_ref[...] = jnp.clip(o + bl_ref[...], -10.0, 10.0)


def _tc(body, out_shape, *args):
    return pl.pallas_call(body, out_shape=out_shape)(*args)


_MESH = plsc.VectorSubcoreMesh(core_axis_name="c", subcore_axis_name="s")
_SC_PARAMS = pltpu.CompilerParams(needs_layout_passes=False)


@functools.partial(
    pl.kernel,
    out_type=jax.ShapeDtypeStruct((NTILES * N,), jnp.float32),
    mesh=_MESH,
    compiler_params=_SC_PARAMS,
    scratch_types=[
        pltpu.VMEM((CHD,), jnp.float32),
        pltpu.VMEM((CHD,), jnp.int32),
        pltpu.VMEM((N,), jnp.float32),
    ],
)
def _sc_deg(ew_hbm, col_hbm, part_hbm, ew_v, col_v, acc_v):
    cid = lax.axis_index("c")
    sid = lax.axis_index("s")
    wid = cid * NS + sid
    base = wid * EPT

    @plsc.parallel_loop(0, N // 16, unroll=4)
    def zero_body(i):
        acc_v[pl.ds(i * 16, 16)] = jnp.zeros((16,), jnp.float32)

    def chunk_body(k, _):
        off = base + k * CHD
        pltpu.sync_copy(ew_hbm.at[pl.ds(off, CHD)], ew_v)
        pltpu.sync_copy(col_hbm.at[pl.ds(off, CHD)], col_v)

        @plsc.parallel_loop(0, CHD // 16, unroll=4)
        def grp(j):
            c16 = col_v[pl.ds(j * 16, 16)]
            w16 = ew_v[pl.ds(j * 16, 16)]
            plsc.addupdate_scatter(acc_v, [c16], w16)

        return _

    lax.fori_loop(0, EPT // CHD, chunk_body, None)
    pltpu.sync_copy(acc_v, part_hbm.at[pl.ds(wid * N, N)])


@functools.partial(
    pl.kernel,
    out_type=jax.ShapeDtypeStruct((NC * H * N,), jnp.float32),
    mesh=_MESH,
    compiler_params=_SC_PARAMS,
    scratch_types=[
        pltpu.VMEM((N,), jnp.int32),
        pltpu.VMEM((FPT * N,), jnp.float32),
        pltpu.VMEM((CH,), jnp.int32),
        pltpu.VMEM((CH,), jnp.int32),
        pltpu.VMEM((CH,), jnp.float32),
        pltpu.VMEM((CH,), jnp.float32),
        pltpu.SemaphoreType.DMA,
        pltpu.SemaphoreType.DMA,
        pltpu.SemaphoreType.DMA,
        pltpu.SemaphoreType.DMA,
        pltpu.SemaphoreType.DMA,
    ],
)
def _sc_scatter(gp_hbm, rc_hbm, w_hbm, p_hbm, g_v, acc_v, rc_v0, rc_v1,
                w_v0, w_v1, rsem0, rsem1, wsem0, wsem1, gsem):
    cid = lax.axis_index("c")
    sid = lax.axis_index("s")
    rsems = (rsem0, rsem1)
    wsems = (wsem0, wsem1)
    rcbufs = (rc_v0, rc_v1)
    wbufs = (w_v0, w_v1)
    gcp = pltpu.async_copy(gp_hbm.at[pl.ds(sid * N, N)], g_v, gsem)

    @plsc.parallel_loop(0, (FPT * N) // 16, unroll=4)
    def zero_body(i):
        acc_v[pl.ds(i * 16, 16)] = jnp.zeros((16,), jnp.float32)

    gcp.wait()
    ebase = cid * HALF

    def issue(k, b):
        pltpu.async_copy(rc_hbm.at[pl.ds(ebase + k * CH, CH)],
                         rcbufs[b], rsems[b])
        pltpu.async_copy(w_hbm.at[pl.ds(ebase + k * CH, CH)],
                         wbufs[b], wsems[b])

    issue(0, 0)
    issue(1, 1)

    def outer(gi, _):
        for b in range(2):
            rcbuf = rcbufs[b]
            wbuf = wbufs[b]
            k = gi * 2 + b
            pltpu.make_async_copy(rc_hbm.at[pl.ds(ebase, CH)],
                                  rcbuf, rsems[b]).wait()
            pltpu.make_async_copy(w_hbm.at[pl.ds(ebase, CH)],
                                  wbuf, wsems[b]).wait()

            @plsc.parallel_loop(0, CH // 16, unroll=8)
            def grp(j):
                rc16 = rcbuf[pl.ds(j * 16, 16)]
                w16 = wbuf[pl.ds(j * 16, 16)]
                r16 = rc16 & 0xFFFF
                c16 = lax.shift_right_logical(rc16, 16)
                vals = plsc.load_gather(g_v, [r16])
                v0 = plsc.bitcast(lax.shift_left(vals, 16), jnp.float32)
                v1 = plsc.bitcast(vals & jnp.int32(-65536), jnp.float32)
                plsc.addupdate_scatter(acc_v, [c16], v0 * w16)
                plsc.addupdate_scatter(acc_v, [c16 + N], v1 * w16)

            @pl.when(k + 2 < NCHUNK)
            def _reissue():
                issue(k + 2, b)

        return _

    lax.fori_loop(0, NCHUNK // 2, outer, None)
    pltpu.sync_copy(acc_v, p_hbm.at[pl.ds(cid * (H * N) + sid * (FPT * N),
                                          FPT * N)])


def kernel(x, edge_index, edge_attr, emb_w, W_feat, b_feat, W_edge, b_edge,
           W_comb, b_comb, W_c0, b_c0, W_c1, b_c1, W_c2, b_c2, gamma, beta,
           W_lin, b_lin):
    f32 = jnp.float32
    row = edge_index[0]
    col = edge_index[1]
    a0 = edge_attr[:, 0].reshape(E // 128, 128)
    a1 = edge_attr[:, 1].reshape(E // 128, 128)
    wpack = jnp.concatenate([W_edge, b_edge[:, None]], axis=0)

    r2 = row.reshape(E // 128, 128)
    c2 = col.reshape(E // 128, 128)
    ew2, rc2, h0t = _tc(
        _prologue_body,
        (jax.ShapeDtypeStruct((E // 128, 128), f32),
         jax.ShapeDtypeStruct((E // 128, 128), jnp.int32),
         jax.ShapeDtypeStruct((H, N), f32)),
        a0, a1, wpack, r2, c2, x, emb_w, W_feat, b_feat.reshape(1, EMB),
        W_comb, b_comb.reshape(H, 1))
    ew = ew2.reshape(E)
    rc = rc2.reshape(E)

    part = _sc_deg(ew, col).reshape(NTILES, N)

    dis, sn, gp = _tc(
        _norm_body,
        (jax.ShapeDtypeStruct((1, N), f32),
         jax.ShapeDtypeStruct((1, N), f32),
         jax.ShapeDtypeStruct((NS, N), jnp.int32)),
        part, h0t)

    ht = h0t
    for (W, b) in ((W_c0, b_c0), (W_c1, b_c1)):
        pflat = _sc_scatter(gp.reshape(NS * N), rc, ew)
        p2 = pflat.reshape(NC, NS, FPT, N)
        ht, gp = _tc(
            _layer_body,
            (jax.ShapeDtypeStruct((H, N), f32),
             jax.ShapeDtypeStruct((NS, N), jnp.int32)),
            p2, ht, dis, sn, W, b.reshape(H, 1), gamma.reshape(H, 1),
            beta.reshape(H, 1))

    pflat = _sc_scatter(gp.reshape(NS * N), rc, ew)
    p2 = pflat.reshape(NC, NS, FPT, N)
    out = _tc(
        _layer_final_body, jax.ShapeDtypeStruct((N, 1), f32),
        p2, ht, dis, sn, W_c2, b_c2.reshape(H, 1), gamma.reshape(H, 1),
        beta.reshape(H, 1), W_lin, b_lin.reshape(1, 1))
    return out
